# baseline (speedup 1.0000x reference)
import jax
import jax.numpy as jnp
from jax import lax
from jax.experimental import pallas as pl
from jax.experimental.pallas import tpu as pltpu
from jax.experimental.pallas import tpu_sc as plsc

NC, NS, L = 2, 16, 16
NW = NC * NS
VOCAB, D, S = 100000, 128, 512
B = 1024
N = B * S
TPW = N // NW
C = 128
NCHUNK = TPW // C
CPS = S // C
RV = 1000


def _t2_body(tok_tab_ref, seg_tab_ref, t2_ref):
    x = tok_tab_ref[...]
    for s in range(2):
        bits = lax.bitcast_convert_type(x + seg_tab_ref[s, :], jnp.int32)
        r16 = lax.shift_right_arithmetic(
            bits + 0x7FFF + lax.bitwise_and(
                lax.shift_right_arithmetic(bits, 16), 1), 16)
        z = r16.reshape(RV, 4, 2, 16)
        t2_ref[:, s, :] = (
            lax.shift_left(z[:, :, 1, :], 16)
            | lax.bitwise_and(z[:, :, 0, :], 0xFFFF)
        ).reshape(RV, D // 2)


def _build_t2(token_table, segment_table):
    return pl.pallas_call(
        _t2_body,
        grid=(VOCAB // RV,),
        in_specs=[
            pl.BlockSpec((RV, D), lambda i: (i, 0)),
            pl.BlockSpec((2, D), lambda i: (0, 0)),
        ],
        out_specs=pl.BlockSpec((RV, 2, D // 2), lambda i: (i, 0, 0)),
        out_shape=jax.ShapeDtypeStruct((VOCAB, 2, D // 2), jnp.int32),
    )(token_table, segment_table)


def _idx2_body(tok_ref, seg_ref, idx_ref):
    idx_ref[...] = tok_ref[...] * 2 + seg_ref[...]


def _build_idx2(token_ids, segment_ids):
    return pl.pallas_call(
        _idx2_body,
        grid=(8,),
        in_specs=[
            pl.BlockSpec((B // 8, S), lambda i: (i, 0)),
            pl.BlockSpec((B // 8, S), lambda i: (i, 0)),
        ],
        out_specs=pl.BlockSpec((B // 8, S), lambda i: (i, 0)),
        out_shape=jax.ShapeDtypeStruct((B, S), jnp.int32),
    )(token_ids, segment_ids)


def _body(idx2_hbm, t2_hbm, pos_hbm, out_hbm,
          base_v, rows0, rows1, out0, out1, idx0, idx1, g0, g1, o0, o1):
    rows = (rows0, rows1)
    outs = (out0, out1)
    idxs = (idx0, idx1)
    gsem = (g0, g1)
    osem = (o0, o1)

    wid = lax.axis_index("s") * NC + lax.axis_index("c")
    wbase = wid * TPW

    pltpu.sync_copy(pos_hbm, base_v)

    def load_chunk(c, p):
        pltpu.sync_copy(idx2_hbm.at[pl.ds(wbase + c * C, C)], idxs[p])

    def start_gather(p):
        pltpu.async_copy(t2_hbm.at[idxs[p]], rows[p], gsem[p])

    def wait_gather(p):
        pltpu.make_async_copy(t2_hbm.at[idxs[p]], rows[p], gsem[p]).wait()

    def compute(c, p):
        p0 = lax.rem(c, CPS) * C
        r = rows[p]
        ob = outs[p]

        @plsc.parallel_loop(0, C, unroll=2)
        def tok_body(t):
            pr = p0 + t
            for h in range(D // 32):
                w = r[t, pl.ds(h * L, L)]
                lo = lax.bitcast_convert_type(lax.shift_left(w, 16), jnp.float32)
                hi = lax.bitcast_convert_type(lax.bitwise_and(w, jnp.int32(-65536)), jnp.float32)
                sl0 = pl.ds(h * 32, L)
                sl1 = pl.ds(h * 32 + L, L)
                ob[t, sl0] = lo + base_v[pr, sl0]
                ob[t, sl1] = hi + base_v[pr, sl1]

    def start_outcopy(c, p):
        pltpu.async_copy(outs[p], out_hbm.at[pl.ds(wbase + c * C, C)], osem[p])

    def wait_outcopy(c, p):
        pltpu.make_async_copy(
            outs[p], out_hbm.at[pl.ds(wbase + c * C, C)], osem[p]).wait()

    load_chunk(0, 0)
    load_chunk(1, 1)
    start_gather(0)
    start_gather(1)

    for i in range(2):
        wait_gather(i)
        compute(i, i)
        start_outcopy(i, i)
        load_chunk(i + 2, i)
        start_gather(i)

    def group(g, carry):
        for k in range(2):
            i = 2 * g + 2 + k
            p = k
            wait_gather(p)
            wait_outcopy(i - 2, p)
            compute(i, p)
            start_outcopy(i, p)
            load_chunk(i + 2, p)
            start_gather(p)
        return carry
    lax.fori_loop(0, (NCHUNK - 4) // 2, group, 0, unroll=False)

    for k in range(2):
        i = NCHUNK - 2 + k
        p = k
        wait_gather(p)
        wait_outcopy(i - 2, p)
        compute(i, p)
        start_outcopy(i, p)

    wait_outcopy(NCHUNK - 2, 0)
    wait_outcopy(NCHUNK - 1, 1)


@jax.jit
def _embed(token_ids, segment_ids, token_table, segment_table, position_table):
    t2 = _build_t2(token_table, segment_table).reshape(2 * VOCAB, D // 2)
    idx2 = _build_idx2(token_ids, segment_ids).reshape(N)
    mesh = plsc.VectorSubcoreMesh(
        core_axis_name="c", subcore_axis_name="s", num_cores=NC, num_subcores=NS)
    f = pl.kernel(
        _body,
        out_type=jax.ShapeDtypeStruct((N, D), jnp.float32),
        mesh=mesh,
        compiler_params=pltpu.CompilerParams(use_tc_tiling_on_sc=False),
        scratch_types=[
            pltpu.VMEM((S, D), jnp.float32),
            pltpu.VMEM((C, D // 2), jnp.int32),
            pltpu.VMEM((C, D // 2), jnp.int32),
            pltpu.VMEM((C, D), jnp.float32),
            pltpu.VMEM((C, D), jnp.float32),
            pltpu.VMEM((C,), jnp.int32),
            pltpu.VMEM((C,), jnp.int32),
            pltpu.SemaphoreType.DMA,
            pltpu.SemaphoreType.DMA,
            pltpu.SemaphoreType.DMA,
            pltpu.SemaphoreType.DMA,
        ],
    )
    return f(idx2, t2, position_table)


def kernel(token_ids, segment_ids, token_table, segment_table, position_table):
    out = _embed(token_ids, segment_ids, token_table, segment_table,
                 position_table)
    return out.reshape(B, S, D)


# speedup vs baseline: 2.9972x; 2.9972x over previous
import jax
import jax.numpy as jnp
from jax import lax
from jax.experimental import pallas as pl
from jax.experimental.pallas import tpu as pltpu
from jax.experimental.pallas import tpu_sc as plsc

NC, NS, L = 2, 16, 16
NW = NC * NS
VOCAB, D, S = 100000, 128, 512
B = 1024
N = B * S
TPW = N // NW
C = 128
NCHUNK = TPW // C
CPS = S // C
RV = 1000


def _t2_body(tok_tab_ref, seg_tab_ref, t2_ref):
    x = tok_tab_ref[...]
    for s in range(2):
        bits = lax.bitcast_convert_type(x + seg_tab_ref[s, :], jnp.int32)
        r16 = lax.shift_right_arithmetic(
            bits + 0x7FFF + lax.bitwise_and(
                lax.shift_right_arithmetic(bits, 16), 1), 16)
        t2_ref[:, s, :] = (
            lax.shift_left(r16[:, D // 2:], 16)
            | lax.bitwise_and(r16[:, :D // 2], 0xFFFF))


def _build_t2(token_table, segment_table):
    return pl.pallas_call(
        _t2_body,
        grid=(VOCAB // RV,),
        in_specs=[
            pl.BlockSpec((RV, D), lambda i: (i, 0)),
            pl.BlockSpec((2, D), lambda i: (0, 0)),
        ],
        out_specs=pl.BlockSpec((RV, 2, D // 2), lambda i: (i, 0, 0)),
        out_shape=jax.ShapeDtypeStruct((VOCAB, 2, D // 2), jnp.int32),
    )(token_table, segment_table)


def _idx2_body(tok_ref, seg_ref, idx_ref):
    idx_ref[...] = tok_ref[...] * 2 + seg_ref[...]


def _build_idx2(token_ids, segment_ids):
    return pl.pallas_call(
        _idx2_body,
        grid=(8,),
        in_specs=[
            pl.BlockSpec((B // 8, S), lambda i: (i, 0)),
            pl.BlockSpec((B // 8, S), lambda i: (i, 0)),
        ],
        out_specs=pl.BlockSpec((B // 8, S), lambda i: (i, 0)),
        out_shape=jax.ShapeDtypeStruct((B, S), jnp.int32),
    )(token_ids, segment_ids)


def _body(idx2_hbm, t2_hbm, pos_hbm, out_hbm,
          base_v, rows0, rows1, out0, out1, idx0, idx1, g0, g1, o0, o1):
    rows = (rows0, rows1)
    outs = (out0, out1)
    idxs = (idx0, idx1)
    gsem = (g0, g1)
    osem = (o0, o1)

    wid = lax.axis_index("s") * NC + lax.axis_index("c")
    wbase = wid * TPW

    pltpu.sync_copy(pos_hbm, base_v)

    def load_chunk(c, p):
        pltpu.sync_copy(idx2_hbm.at[pl.ds(wbase + c * C, C)], idxs[p])

    def start_gather(p):
        pltpu.async_copy(t2_hbm.at[idxs[p]], rows[p], gsem[p])

    def wait_gather(p):
        pltpu.make_async_copy(t2_hbm.at[idxs[p]], rows[p], gsem[p]).wait()

    def compute(c, p):
        p0 = lax.rem(c, CPS) * C
        r = rows[p]
        ob = outs[p]

        @plsc.parallel_loop(0, C, unroll=2)
        def tok_body(t):
            pr = p0 + t
            for h in range(D // 32):
                w = r[t, pl.ds(h * L, L)]
                lo = lax.bitcast_convert_type(lax.shift_left(w, 16), jnp.float32)
                hi = lax.bitcast_convert_type(lax.bitwise_and(w, jnp.int32(-65536)), jnp.float32)
                sl0 = pl.ds(h * L, L)
                sl1 = pl.ds(D // 2 + h * L, L)
                ob[t, sl0] = lo + base_v[pr, sl0]
                ob[t, sl1] = hi + base_v[pr, sl1]

    def start_outcopy(c, p):
        pltpu.async_copy(outs[p], out_hbm.at[pl.ds(wbase + c * C, C)], osem[p])

    def wait_outcopy(c, p):
        pltpu.make_async_copy(
            outs[p], out_hbm.at[pl.ds(wbase + c * C, C)], osem[p]).wait()

    load_chunk(0, 0)
    load_chunk(1, 1)
    start_gather(0)
    start_gather(1)

    for i in range(2):
        wait_gather(i)
        compute(i, i)
        start_outcopy(i, i)
        load_chunk(i + 2, i)
        start_gather(i)

    def group(g, carry):
        for k in range(2):
            i = 2 * g + 2 + k
            p = k
            wait_gather(p)
            wait_outcopy(i - 2, p)
            compute(i, p)
            start_outcopy(i, p)
            load_chunk(i + 2, p)
            start_gather(p)
        return carry
    lax.fori_loop(0, (NCHUNK - 4) // 2, group, 0, unroll=False)

    for k in range(2):
        i = NCHUNK - 2 + k
        p = k
        wait_gather(p)
        wait_outcopy(i - 2, p)
        compute(i, p)
        start_outcopy(i, p)

    wait_outcopy(NCHUNK - 2, 0)
    wait_outcopy(NCHUNK - 1, 1)


@jax.jit
def _embed(token_ids, segment_ids, token_table, segment_table, position_table):
    t2 = _build_t2(token_table, segment_table).reshape(2 * VOCAB, D // 2)
    idx2 = _build_idx2(token_ids, segment_ids).reshape(N)
    mesh = plsc.VectorSubcoreMesh(
        core_axis_name="c", subcore_axis_name="s", num_cores=NC, num_subcores=NS)
    f = pl.kernel(
        _body,
        out_type=jax.ShapeDtypeStruct((N, D), jnp.float32),
        mesh=mesh,
        compiler_params=pltpu.CompilerParams(use_tc_tiling_on_sc=False),
        scratch_types=[
            pltpu.VMEM((S, D), jnp.float32),
            pltpu.VMEM((C, D // 2), jnp.int32),
            pltpu.VMEM((C, D // 2), jnp.int32),
            pltpu.VMEM((C, D), jnp.float32),
            pltpu.VMEM((C, D), jnp.float32),
            pltpu.VMEM((C,), jnp.int32),
            pltpu.VMEM((C,), jnp.int32),
            pltpu.SemaphoreType.DMA,
            pltpu.SemaphoreType.DMA,
            pltpu.SemaphoreType.DMA,
            pltpu.SemaphoreType.DMA,
        ],
    )
    return f(idx2, t2, position_table)


def kernel(token_ids, segment_ids, token_table, segment_table, position_table):
    out = _embed(token_ids, segment_ids, token_table, segment_table,
                 position_table)
    return out.reshape(B, S, D)


# speedup vs baseline: 5.2276x; 1.7442x over previous
import jax
import jax.numpy as jnp
from jax import lax
from jax.experimental import pallas as pl
from jax.experimental.pallas import tpu as pltpu
from jax.experimental.pallas import tpu_sc as plsc

NC, NS, L = 2, 16, 16
NW = NC * NS
VOCAB, D, S = 100000, 128, 512
B = 1024
N = B * S
TPW = N // NW
C = 128
NCHUNK = TPW // C
CPS = S // C
RV = 1000


def _t2_body(tok_tab_ref, seg_tab_ref, t2_ref):
    x = tok_tab_ref[...]
    for s in range(2):
        bits = lax.bitcast_convert_type(x + seg_tab_ref[s, :], jnp.int32)
        r16 = lax.shift_right_arithmetic(
            bits + 0x7FFF + lax.bitwise_and(
                lax.shift_right_arithmetic(bits, 16), 1), 16)
        t2_ref[:, pl.ds(s * (D // 2), D // 2)] = (
            lax.shift_left(r16[:, D // 2:], 16)
            | lax.bitwise_and(r16[:, :D // 2], 0xFFFF))


def _build_t2(token_table, segment_table):
    return pl.pallas_call(
        _t2_body,
        grid=(VOCAB // RV,),
        in_specs=[
            pl.BlockSpec((RV, D), lambda i: (i, 0)),
            pl.BlockSpec((2, D), lambda i: (0, 0)),
        ],
        out_specs=pl.BlockSpec((RV, D), lambda i: (i, 0)),
        out_shape=jax.ShapeDtypeStruct((VOCAB, D), jnp.int32),
    )(token_table, segment_table)


def _idx2_body(tok_ref, seg_ref, idx_ref):
    idx_ref[...] = tok_ref[...] * 2 + seg_ref[...]


def _build_idx2(token_ids, segment_ids):
    return pl.pallas_call(
        _idx2_body,
        grid=(8,),
        in_specs=[
            pl.BlockSpec((B // 8, S), lambda i: (i, 0)),
            pl.BlockSpec((B // 8, S), lambda i: (i, 0)),
        ],
        out_specs=pl.BlockSpec((B // 8, S), lambda i: (i, 0)),
        out_shape=jax.ShapeDtypeStruct((B, S), jnp.int32),
    )(token_ids, segment_ids)


def _body(idx2_hbm, t2_hbm, pos_hbm, out_hbm,
          base_v, rows0, rows1, out0, out1, idx0, idx1, g0, g1, o0, o1):
    rows = (rows0, rows1)
    outs = (out0, out1)
    idxs = (idx0, idx1)
    gsem = (g0, g1)
    osem = (o0, o1)

    wid = lax.axis_index("s") * NC + lax.axis_index("c")
    wbase = wid * TPW

    pltpu.sync_copy(pos_hbm, base_v)

    def load_chunk(c, p):
        pltpu.sync_copy(idx2_hbm.at[pl.ds(wbase + c * C, C)], idxs[p])

    def start_gather(p):
        pltpu.async_copy(t2_hbm.at[idxs[p]], rows[p], gsem[p])

    def wait_gather(p):
        pltpu.make_async_copy(t2_hbm.at[idxs[p]], rows[p], gsem[p]).wait()

    def compute(c, p):
        p0 = lax.rem(c, CPS) * C
        r = rows[p]
        ob = outs[p]

        @plsc.parallel_loop(0, C, unroll=2)
        def tok_body(t):
            pr = p0 + t
            for h in range(D // 32):
                w = r[t, pl.ds(h * L, L)]
                lo = lax.bitcast_convert_type(lax.shift_left(w, 16), jnp.float32)
                hi = lax.bitcast_convert_type(lax.bitwise_and(w, jnp.int32(-65536)), jnp.float32)
                sl0 = pl.ds(h * L, L)
                sl1 = pl.ds(D // 2 + h * L, L)
                ob[t, sl0] = lo + base_v[pr, sl0]
                ob[t, sl1] = hi + base_v[pr, sl1]

    def start_outcopy(c, p):
        pltpu.async_copy(outs[p], out_hbm.at[pl.ds(wbase + c * C, C)], osem[p])

    def wait_outcopy(c, p):
        pltpu.make_async_copy(
            outs[p], out_hbm.at[pl.ds(wbase + c * C, C)], osem[p]).wait()

    load_chunk(0, 0)
    load_chunk(1, 1)
    start_gather(0)
    start_gather(1)

    for i in range(2):
        wait_gather(i)
        compute(i, i)
        start_outcopy(i, i)
        load_chunk(i + 2, i)
        start_gather(i)

    def group(g, carry):
        for k in range(2):
            i = 2 * g + 2 + k
            p = k
            wait_gather(p)
            wait_outcopy(i - 2, p)
            compute(i, p)
            start_outcopy(i, p)
            load_chunk(i + 2, p)
            start_gather(p)
        return carry
    lax.fori_loop(0, (NCHUNK - 4) // 2, group, 0, unroll=False)

    for k in range(2):
        i = NCHUNK - 2 + k
        p = k
        wait_gather(p)
        wait_outcopy(i - 2, p)
        compute(i, p)
        start_outcopy(i, p)

    wait_outcopy(NCHUNK - 2, 0)
    wait_outcopy(NCHUNK - 1, 1)


@jax.jit
def _embed(token_ids, segment_ids, token_table, segment_table, position_table):
    t2 = _build_t2(token_table, segment_table).reshape(2 * VOCAB, D // 2)
    idx2 = _build_idx2(token_ids, segment_ids).reshape(N)
    mesh = plsc.VectorSubcoreMesh(
        core_axis_name="c", subcore_axis_name="s", num_cores=NC, num_subcores=NS)
    f = pl.kernel(
        _body,
        out_type=jax.ShapeDtypeStruct((N, D), jnp.float32),
        mesh=mesh,
        compiler_params=pltpu.CompilerParams(use_tc_tiling_on_sc=False),
        scratch_types=[
            pltpu.VMEM((S, D), jnp.float32),
            pltpu.VMEM((C, D // 2), jnp.int32),
            pltpu.VMEM((C, D // 2), jnp.int32),
            pltpu.VMEM((C, D), jnp.float32),
            pltpu.VMEM((C, D), jnp.float32),
            pltpu.VMEM((C,), jnp.int32),
            pltpu.VMEM((C,), jnp.int32),
            pltpu.SemaphoreType.DMA,
            pltpu.SemaphoreType.DMA,
            pltpu.SemaphoreType.DMA,
            pltpu.SemaphoreType.DMA,
        ],
    )
    return f(idx2, t2, position_table)


def kernel(token_ids, segment_ids, token_table, segment_table, position_table):
    out = _embed(token_ids, segment_ids, token_table, segment_table,
                 position_table)
    return out.reshape(B, S, D)


# speedup vs baseline: 5.2363x; 1.0017x over previous
import jax
import jax.numpy as jnp
from jax import lax
from jax.experimental import pallas as pl
from jax.experimental.pallas import tpu as pltpu
from jax.experimental.pallas import tpu_sc as plsc

NC, NS, L = 2, 16, 16
NW = NC * NS
VOCAB, D, S = 100000, 128, 512
B = 1024
N = B * S
TPW = N // NW
C = 128
NCHUNK = TPW // C
CPS = S // C
RV = 1000


def _t2_body(tok_tab_ref, seg_tab_ref, t2_ref):
    x = tok_tab_ref[...]
    for s in range(2):
        bits = lax.bitcast_convert_type(x + seg_tab_ref[s, :], jnp.int32)
        r16 = lax.shift_right_arithmetic(bits + 0x8000, 16)
        t2_ref[:, pl.ds(s * (D // 2), D // 2)] = (
            lax.shift_left(r16[:, D // 2:], 16)
            | lax.bitwise_and(r16[:, :D // 2], 0xFFFF))


def _build_t2(token_table, segment_table):
    return pl.pallas_call(
        _t2_body,
        grid=(VOCAB // RV,),
        in_specs=[
            pl.BlockSpec((RV, D), lambda i: (i, 0)),
            pl.BlockSpec((2, D), lambda i: (0, 0)),
        ],
        out_specs=pl.BlockSpec((RV, D), lambda i: (i, 0)),
        out_shape=jax.ShapeDtypeStruct((VOCAB, D), jnp.int32),
    )(token_table, segment_table)


def _idx2_body(tok_ref, seg_ref, idx_ref):
    idx_ref[...] = tok_ref[...] * 2 + seg_ref[...]


def _build_idx2(token_ids, segment_ids):
    return pl.pallas_call(
        _idx2_body,
        grid=(8,),
        in_specs=[
            pl.BlockSpec((B // 8, S), lambda i: (i, 0)),
            pl.BlockSpec((B // 8, S), lambda i: (i, 0)),
        ],
        out_specs=pl.BlockSpec((B // 8, S), lambda i: (i, 0)),
        out_shape=jax.ShapeDtypeStruct((B, S), jnp.int32),
    )(token_ids, segment_ids)


def _body(idx2_hbm, t2_hbm, pos_hbm, out_hbm,
          base_v, rows0, rows1, out0, out1, idx0, idx1, g0, g1, o0, o1):
    rows = (rows0, rows1)
    outs = (out0, out1)
    idxs = (idx0, idx1)
    gsem = (g0, g1)
    osem = (o0, o1)

    wid = lax.axis_index("s") * NC + lax.axis_index("c")
    wbase = wid * TPW

    pltpu.sync_copy(pos_hbm, base_v)

    def load_chunk(c, p):
        pltpu.sync_copy(idx2_hbm.at[pl.ds(wbase + c * C, C)], idxs[p])

    def start_gather(p):
        pltpu.async_copy(t2_hbm.at[idxs[p]], rows[p], gsem[p])

    def wait_gather(p):
        pltpu.make_async_copy(t2_hbm.at[idxs[p]], rows[p], gsem[p]).wait()

    def compute(c, p):
        p0 = lax.rem(c, CPS) * C
        r = rows[p]
        ob = outs[p]

        @plsc.parallel_loop(0, C, unroll=2)
        def tok_body(t):
            pr = p0 + t
            for h in range(D // 32):
                w = r[t, pl.ds(h * L, L)]
                lo = lax.bitcast_convert_type(lax.shift_left(w, 16), jnp.float32)
                hi = lax.bitcast_convert_type(lax.bitwise_and(w, jnp.int32(-65536)), jnp.float32)
                sl0 = pl.ds(h * L, L)
                sl1 = pl.ds(D // 2 + h * L, L)
                ob[t, sl0] = lo + base_v[pr, sl0]
                ob[t, sl1] = hi + base_v[pr, sl1]

    def start_outcopy(c, p):
        pltpu.async_copy(outs[p], out_hbm.at[pl.ds(wbase + c * C, C)], osem[p])

    def wait_outcopy(c, p):
        pltpu.make_async_copy(
            outs[p], out_hbm.at[pl.ds(wbase + c * C, C)], osem[p]).wait()

    load_chunk(0, 0)
    load_chunk(1, 1)
    start_gather(0)
    start_gather(1)

    for i in range(2):
        wait_gather(i)
        compute(i, i)
        start_outcopy(i, i)
        load_chunk(i + 2, i)
        start_gather(i)

    def group(g, carry):
        for k in range(2):
            i = 2 * g + 2 + k
            p = k
            wait_gather(p)
            wait_outcopy(i - 2, p)
            compute(i, p)
            start_outcopy(i, p)
            load_chunk(i + 2, p)
            start_gather(p)
        return carry
    lax.fori_loop(0, (NCHUNK - 4) // 2, group, 0, unroll=False)

    for k in range(2):
        i = NCHUNK - 2 + k
        p = k
        wait_gather(p)
        wait_outcopy(i - 2, p)
        compute(i, p)
        start_outcopy(i, p)

    wait_outcopy(NCHUNK - 2, 0)
    wait_outcopy(NCHUNK - 1, 1)


@jax.jit
def _embed(token_ids, segment_ids, token_table, segment_table, position_table):
    t2 = _build_t2(token_table, segment_table).reshape(2 * VOCAB, D // 2)
    idx2 = _build_idx2(token_ids, segment_ids).reshape(N)
    mesh = plsc.VectorSubcoreMesh(
        core_axis_name="c", subcore_axis_name="s", num_cores=NC, num_subcores=NS)
    f = pl.kernel(
        _body,
        out_type=jax.ShapeDtypeStruct((N, D), jnp.float32),
        mesh=mesh,
        compiler_params=pltpu.CompilerParams(use_tc_tiling_on_sc=False),
        scratch_types=[
            pltpu.VMEM((S, D), jnp.float32),
            pltpu.VMEM((C, D // 2), jnp.int32),
            pltpu.VMEM((C, D // 2), jnp.int32),
            pltpu.VMEM((C, D), jnp.float32),
            pltpu.VMEM((C, D), jnp.float32),
            pltpu.VMEM((C,), jnp.int32),
            pltpu.VMEM((C,), jnp.int32),
            pltpu.SemaphoreType.DMA,
            pltpu.SemaphoreType.DMA,
            pltpu.SemaphoreType.DMA,
            pltpu.SemaphoreType.DMA,
        ],
    )
    return f(idx2, t2, position_table)


def kernel(token_ids, segment_ids, token_table, segment_table, position_table):
    out = _embed(token_ids, segment_ids, token_table, segment_table,
                 position_table)
    return out.reshape(B, S, D)
